# baseline (speedup 1.0000x reference)
import functools

import jax
import jax.numpy as jnp
from jax import lax
from jax.experimental import pallas as pl
from jax.experimental.pallas import tpu as pltpu
from jax.experimental.pallas import tpu_sc as plsc

B = 16384
V = 1000000
D = 16
NC = 2
NS = 16
NW = NC * NS
NPW = B // NW
CH = 128
NCH = NPW // CH
L = 16


def _sc_body(uidx_hbm, iidx_hbm, emb_u_hbm, emb_i_hbm, bias_u_hbm,
             bias_i_hbm, gb_hbm, out_hbm,
             uidx_v, iidx_v, eu_v, ei_v, bu_v, bi_v, gb_v, out_v, sem):
    wid = lax.axis_index("s") * NC + lax.axis_index("c")
    rowbase = wid * NCH

    pltpu.sync_copy(uidx_hbm.at[pl.ds(rowbase, NCH)], uidx_v)
    pltpu.sync_copy(iidx_hbm.at[pl.ds(rowbase, NCH)], iidx_v)
    pltpu.sync_copy(gb_hbm, gb_v)

    copies = []
    for c in range(NCH):
        dst = pl.ds(c * CH, CH)
        copies.append(pltpu.async_copy(emb_u_hbm.at[uidx_v.at[c]], eu_v.at[dst], sem))
        copies.append(pltpu.async_copy(emb_i_hbm.at[iidx_v.at[c]], ei_v.at[dst], sem))
        copies.append(pltpu.async_copy(bias_u_hbm.at[uidx_v.at[c]], bu_v.at[dst], sem))
        copies.append(pltpu.async_copy(bias_i_hbm.at[iidx_v.at[c]], bi_v.at[dst], sem))
    for cp in copies:
        cp.wait()

    gb = gb_v[...]
    dsplats = [jnp.full((L,), d, dtype=jnp.int32) for d in range(D)]
    for i in range(NPW // L):
        rows = jnp.arange(i * L, (i + 1) * L, dtype=jnp.int32)
        acc = bu_v[pl.ds(i * L, L)] + bi_v[pl.ds(i * L, L)] + gb
        for d in range(D):
            gu = plsc.load_gather(eu_v, [rows, dsplats[d]])
            gi = plsc.load_gather(ei_v, [rows, dsplats[d]])
            acc = acc + gu * gi
        out_v[pl.ds(i * L, L)] = jnp.minimum(jnp.maximum(acc, 1.0), 5.0)

    pltpu.sync_copy(out_v, out_hbm.at[pl.ds(wid * NPW, NPW)])


@jax.jit
def _funk_svd_sc(uidx, iidx, emb_u, emb_i, bias_u, bias_i, gb16):
    mesh = plsc.VectorSubcoreMesh(
        core_axis_name="c", subcore_axis_name="s",
        num_cores=NC, num_subcores=NS)
    run = pl.kernel(
        _sc_body,
        out_type=jax.ShapeDtypeStruct((B,), jnp.float32),
        mesh=mesh,
        scratch_types=[
            pltpu.VMEM((NCH, CH), jnp.int32),
            pltpu.VMEM((NCH, CH), jnp.int32),
            pltpu.VMEM((NPW, D), jnp.float32),
            pltpu.VMEM((NPW, D), jnp.float32),
            pltpu.VMEM((NPW,), jnp.float32),
            pltpu.VMEM((NPW,), jnp.float32),
            pltpu.VMEM((L,), jnp.float32),
            pltpu.VMEM((NPW,), jnp.float32),
            pltpu.SemaphoreType.DMA,
        ],
        compiler_params=pltpu.CompilerParams(
            needs_layout_passes=False, use_tc_tiling_on_sc=False),
    )
    return run(uidx, iidx, emb_u, emb_i, bias_u, bias_i, gb16)


def kernel(user_idx, item_idx, emb_u, emb_i, bias_u, bias_i, global_bias):
    uidx = user_idx.astype(jnp.int32).reshape(B // CH, CH)
    iidx = item_idx.astype(jnp.int32).reshape(B // CH, CH)
    gb16 = jnp.broadcast_to(global_bias.astype(jnp.float32), (L,))
    return _funk_svd_sc(uidx, iidx, emb_u, emb_i, bias_u, bias_i, gb16)


# speedup vs baseline: 1.0047x; 1.0047x over previous
import functools

import jax
import jax.numpy as jnp
from jax import lax
from jax.experimental import pallas as pl
from jax.experimental.pallas import tpu as pltpu
from jax.experimental.pallas import tpu_sc as plsc

B = 16384
V = 1000000
D = 16
NC = 2
NS = 16
NW = NC * NS
NPW = B // NW
CH = 128
NCH = NPW // CH
L = 16


def _sc_body(uidx_hbm, iidx_hbm, emb_u_hbm, emb_i_hbm, bias_u_hbm,
             bias_i_hbm, gb_hbm, out_hbm,
             uidx_v, iidx_v, eu_v, ei_v, bu_v, bi_v, gb_v, out_v, sem):
    wid = lax.axis_index("s") * NC + lax.axis_index("c")
    rowbase = wid * NCH

    pltpu.sync_copy(uidx_hbm.at[pl.ds(rowbase, NCH)], uidx_v)
    pltpu.sync_copy(iidx_hbm.at[pl.ds(rowbase, NCH)], iidx_v)
    pltpu.sync_copy(gb_hbm, gb_v)

    copies = []
    for c in range(NCH):
        dst = pl.ds(c * CH, CH)
        copies.append(pltpu.async_copy(emb_u_hbm.at[uidx_v.at[c]], eu_v.at[dst], sem))
        copies.append(pltpu.async_copy(emb_i_hbm.at[iidx_v.at[c]], ei_v.at[dst], sem))
        copies.append(pltpu.async_copy(bias_u_hbm.at[uidx_v.at[c]], bu_v.at[dst], sem))
        copies.append(pltpu.async_copy(bias_i_hbm.at[iidx_v.at[c]], bi_v.at[dst], sem))
    for cp in copies:
        cp.wait()

    gb = gb_v[...]
    dsplats = [jnp.full((L,), d, dtype=jnp.int32) for d in range(D)]
    iota = lax.broadcasted_iota(jnp.int32, (L,), 0)

    def block(i, _):
        base = i * L
        rows = base + iota
        acc = bu_v[pl.ds(base, L)] + bi_v[pl.ds(base, L)] + gb
        for d in range(D):
            gu = plsc.load_gather(eu_v, [rows, dsplats[d]])
            gi = plsc.load_gather(ei_v, [rows, dsplats[d]])
            acc = acc + gu * gi
        out_v[pl.ds(base, L)] = jnp.minimum(jnp.maximum(acc, 1.0), 5.0)
        return 0

    lax.fori_loop(0, NPW // L, block, 0)

    pltpu.sync_copy(out_v, out_hbm.at[pl.ds(wid * NPW, NPW)])


@jax.jit
def _funk_svd_sc(uidx, iidx, emb_u, emb_i, bias_u, bias_i, gb16):
    mesh = plsc.VectorSubcoreMesh(
        core_axis_name="c", subcore_axis_name="s",
        num_cores=NC, num_subcores=NS)
    run = pl.kernel(
        _sc_body,
        out_type=jax.ShapeDtypeStruct((B,), jnp.float32),
        mesh=mesh,
        scratch_types=[
            pltpu.VMEM((NCH, CH), jnp.int32),
            pltpu.VMEM((NCH, CH), jnp.int32),
            pltpu.VMEM((NPW, D), jnp.float32),
            pltpu.VMEM((NPW, D), jnp.float32),
            pltpu.VMEM((NPW,), jnp.float32),
            pltpu.VMEM((NPW,), jnp.float32),
            pltpu.VMEM((L,), jnp.float32),
            pltpu.VMEM((NPW,), jnp.float32),
            pltpu.SemaphoreType.DMA,
        ],
        compiler_params=pltpu.CompilerParams(
            needs_layout_passes=False, use_tc_tiling_on_sc=False),
    )
    return run(uidx, iidx, emb_u, emb_i, bias_u, bias_i, gb16)


def kernel(user_idx, item_idx, emb_u, emb_i, bias_u, bias_i, global_bias):
    uidx = user_idx.astype(jnp.int32).reshape(B // CH, CH)
    iidx = item_idx.astype(jnp.int32).reshape(B // CH, CH)
    gb16 = jnp.broadcast_to(global_bias.astype(jnp.float32), (L,))
    return _funk_svd_sc(uidx, iidx, emb_u, emb_i, bias_u, bias_i, gb16)
